# baseline (speedup 1.0000x reference)
import functools

import jax
import jax.numpy as jnp
from jax import lax
from jax.experimental import pallas as pl
from jax.experimental.pallas import tpu as pltpu


_BLOCK_ROWS = 32768
_NBUF = 4


def _make_tc_ring(rows: int, d: int):
    block = _BLOCK_ROWS
    while rows % block:
        block //= 2
    steps = rows // block
    nbuf = min(_NBUF, steps)

    def body(w_ref, o_ref, buf, sem):
        buf[...] = jnp.broadcast_to(w_ref[...], buf.shape)

        for t in range(nbuf):
            pltpu.make_async_copy(
                buf, o_ref.at[pl.ds(t * block, block)], sem
            ).start()

        def ring(t, carry):
            pltpu.make_async_copy(buf, o_ref.at[pl.ds(0, block)], sem).wait()
            pltpu.make_async_copy(
                buf, o_ref.at[pl.ds(t * block, block)], sem
            ).start()
            return carry

        lax.fori_loop(nbuf, steps, ring, 0)

        for _ in range(nbuf):
            pltpu.make_async_copy(buf, o_ref.at[pl.ds(0, block)], sem).wait()

    return pl.pallas_call(
        body,
        in_specs=[pl.BlockSpec(memory_space=pltpu.VMEM)],
        out_specs=pl.BlockSpec(memory_space=pl.ANY),
        out_shape=jax.ShapeDtypeStruct((rows, d), jnp.float32),
        scratch_shapes=[
            pltpu.VMEM((block, d), jnp.float32),
            pltpu.SemaphoreType.DMA,
        ],
    )


def kernel(input, weight):
    B, H = input.shape
    _, D = weight.shape
    out = _make_tc_ring(B * H, D)(weight)
    return out.reshape(B, H, D)


# speedup vs baseline: 1.0006x; 1.0006x over previous
import functools

import jax
import jax.numpy as jnp
from jax import lax
from jax.experimental import pallas as pl
from jax.experimental.pallas import tpu as pltpu


_BLOCK_ROWS = 8192
_NBUF = 8


def _make_tc_ring(rows: int, d: int):
    block = _BLOCK_ROWS
    while rows % block:
        block //= 2
    steps = rows // block
    nbuf = min(_NBUF, steps)

    def body(w_ref, o_ref, buf, sem):
        buf[...] = jnp.broadcast_to(w_ref[...], buf.shape)

        for t in range(nbuf):
            pltpu.make_async_copy(
                buf, o_ref.at[pl.ds(t * block, block)], sem
            ).start()

        def ring(t, carry):
            pltpu.make_async_copy(buf, o_ref.at[pl.ds(0, block)], sem).wait()
            pltpu.make_async_copy(
                buf, o_ref.at[pl.ds(t * block, block)], sem
            ).start()
            return carry

        lax.fori_loop(nbuf, steps, ring, 0)

        for _ in range(nbuf):
            pltpu.make_async_copy(buf, o_ref.at[pl.ds(0, block)], sem).wait()

    return pl.pallas_call(
        body,
        in_specs=[pl.BlockSpec(memory_space=pltpu.VMEM)],
        out_specs=pl.BlockSpec(memory_space=pl.ANY),
        out_shape=jax.ShapeDtypeStruct((rows, d), jnp.float32),
        scratch_shapes=[
            pltpu.VMEM((block, d), jnp.float32),
            pltpu.SemaphoreType.DMA,
        ],
    )


def kernel(input, weight):
    B, H = input.shape
    _, D = weight.shape
    out = _make_tc_ring(B * H, D)(weight)
    return out.reshape(B, H, D)
